# Initial kernel scaffold; baseline (speedup 1.0000x reference)
import jax
import jax.numpy as jnp
from jax.experimental import pallas as pl


def kernel(x, Q, P, WqT, WkT, Wgcn, bgcn):
    raise NotImplementedError("write your pallas kernel here")


# speedup vs baseline: 1.2121x; 1.2121x over previous
import functools
import jax
import jax.numpy as jnp
from jax import lax
from jax.experimental import pallas as pl
from jax.experimental.pallas import tpu as pltpu

_BT = 8
_G = 4


def _fused_kernel(x_ref, qt_ref, p_ref, pt_ref, wqk_ref, gw_ref, gb_ref,
                  bias_ref, r2n_ref, attn_ref, *, scale, k_order, dq, bt,
                  r_dim, g):
    rows = x_ref.shape[0] // g
    for j in range(g):
        x_blk = x_ref[j * rows:(j + 1) * rows, :]
        x_filt = jnp.maximum(qt_ref[...] * x_blk, 0.0)
        x_reg = jnp.dot(p_ref[...], x_filt,
                        preferred_element_type=jnp.float32)

        qk = jnp.dot(x_reg, wqk_ref[...],
                     preferred_element_type=jnp.float32)
        q = qk[:, :dq]
        k = qk[:, dq:]
        dots = lax.dot_general(q, k, (((1,), (1,)), ((), ())),
                               preferred_element_type=jnp.float32)
        if scale != 1.0:
            dots = dots * scale
        dots = dots + bias_ref[...]
        m = jnp.max(dots, axis=-1, keepdims=True)
        e = jnp.exp(dots - m)
        attn = e * pl.reciprocal(jnp.sum(e, axis=-1, keepdims=True),
                                 approx=True)

        for b in range(bt):
            attn_ref[j * bt + b] = attn[b * r_dim:(b + 1) * r_dim,
                                        b * r_dim:(b + 1) * r_dim]

        h = x_reg
        out = jnp.dot(h, gw_ref[0], preferred_element_type=jnp.float32)
        for kk in range(1, k_order):
            h = jnp.dot(attn, h, preferred_element_type=jnp.float32)
            out = out + jnp.dot(h, gw_ref[kk],
                                preferred_element_type=jnp.float32)
        out = jnp.maximum(out + gb_ref[...], 0.0)

        r2n_ref[j * rows:(j + 1) * rows, :] = jnp.dot(
            pt_ref[...], out, preferred_element_type=jnp.float32)


def kernel(x, Q, P, WqT, WkT, Wgcn, bgcn):
    B, N, D = x.shape
    R = P.shape[0]
    K, _, reg_dim = Wgcn.shape
    Dq = WqT.shape[1]

    bt = _BT
    if B % bt or (bt * N) % 8:
        bt = B
    g = next((gg for gg in (_G, 2, 1) if B % (bt * gg) == 0), 1)
    S = B // (bt * g)

    x_flat = x.reshape(B * N, D)
    Q_tile = jnp.tile(Q, (bt, 1))
    eye_bt = jnp.eye(bt, dtype=jnp.float32)
    P_blk = jnp.kron(eye_bt, P.astype(jnp.float32))
    PT_blk = P_blk.T
    Wqk = jnp.concatenate([WqT, WkT], axis=1)
    blk_mask = jnp.kron(eye_bt, jnp.ones((R, R), jnp.float32))
    bias = jnp.where(blk_mask > 0.5, 0.0, -1e30).astype(jnp.float32)

    kernel_fn = functools.partial(_fused_kernel, scale=1.0, k_order=K,
                                  dq=Dq, bt=bt, r_dim=R, g=g)

    out_shapes = (
        jax.ShapeDtypeStruct((B * N, reg_dim), jnp.float32),
        jax.ShapeDtypeStruct((B, R, R), jnp.float32),
    )

    grid_spec = pltpu.PrefetchScalarGridSpec(
        num_scalar_prefetch=0,
        grid=(S,),
        in_specs=[
            pl.BlockSpec((g * bt * N, D), lambda i: (i, 0)),
            pl.BlockSpec((bt * N, D), lambda i: (0, 0)),
            pl.BlockSpec((bt * R, bt * N), lambda i: (0, 0)),
            pl.BlockSpec((bt * N, bt * R), lambda i: (0, 0)),
            pl.BlockSpec((D, 2 * Dq), lambda i: (0, 0)),
            pl.BlockSpec((K, D, reg_dim), lambda i: (0, 0, 0)),
            pl.BlockSpec((1, reg_dim), lambda i: (0, 0)),
            pl.BlockSpec((bt * R, bt * R), lambda i: (0, 0)),
        ],
        out_specs=[
            pl.BlockSpec((g * bt * N, reg_dim), lambda i: (i, 0)),
            pl.BlockSpec((g * bt, R, R), lambda i: (i, 0, 0)),
        ],
    )

    r2n_flat, A_reg = pl.pallas_call(
        kernel_fn,
        grid_spec=grid_spec,
        out_shape=out_shapes,
        compiler_params=pltpu.CompilerParams(
            dimension_semantics=("parallel",)),
    )(x_flat, Q_tile, P_blk, PT_blk, Wqk, Wgcn, bgcn, bias)

    return r2n_flat.reshape(B, N, reg_dim), A_reg


# speedup vs baseline: 1.9632x; 1.6196x over previous
import functools
import jax
import jax.numpy as jnp
from jax import lax
from jax.experimental import pallas as pl
from jax.experimental.pallas import tpu as pltpu

_BT = 8
_G = 4


def _fused_kernel(x_ref, qt_ref, p_ref, pt_ref, wqk_ref, gw_ref, gb_ref,
                  bias_ref, r2n_ref, attn_ref, *, scale, k_order, dq, bt,
                  r_dim, g):
    rows = x_ref.shape[0] // g
    bf = jnp.bfloat16
    p = p_ref[...]
    pt = pt_ref[...]
    wqk = wqk_ref[...]
    qt = qt_ref[...]
    bias = bias_ref[...]
    gb = gb_ref[...]
    gr = range(g)

    xf = [jnp.maximum(qt * x_ref[j * rows:(j + 1) * rows, :],
                      0.0).astype(bf) for j in gr]
    xr = [jnp.dot(p, xf[j], preferred_element_type=jnp.float32) for j in gr]
    xrb = [v.astype(bf) for v in xr]

    qk = [jnp.dot(xrb[j], wqk, preferred_element_type=jnp.float32)
          for j in gr]
    dots = [lax.dot_general(qk[j][:, :dq].astype(bf),
                            qk[j][:, dq:].astype(bf),
                            (((1,), (1,)), ((), ())),
                            preferred_element_type=jnp.float32)
            for j in gr]
    if scale != 1.0:
        dots = [d * scale for d in dots]
    dots = [d + bias for d in dots]
    mx = [jnp.max(d, axis=-1, keepdims=True) for d in dots]
    ex = [jnp.exp(dots[j] - mx[j]) for j in gr]
    attn = [ex[j] * pl.reciprocal(jnp.sum(ex[j], axis=-1, keepdims=True),
                                  approx=True) for j in gr]

    for j in gr:
        for b in range(bt):
            attn_ref[j * bt + b] = attn[j][b * r_dim:(b + 1) * r_dim,
                                          b * r_dim:(b + 1) * r_dim]

    attnb = [a.astype(bf) for a in attn]
    h = xrb
    out = [jnp.dot(h[j], gw_ref[0], preferred_element_type=jnp.float32)
           for j in gr]
    for kk in range(1, k_order):
        h = [jnp.dot(attnb[j], h[j],
                     preferred_element_type=jnp.float32).astype(bf)
             for j in gr]
        out = [out[j] + jnp.dot(h[j], gw_ref[kk],
                                preferred_element_type=jnp.float32)
               for j in gr]
    outb = [jnp.maximum(out[j] + gb, 0.0).astype(bf) for j in gr]

    for j in gr:
        r2n_ref[j * rows:(j + 1) * rows, :] = jnp.dot(
            pt, outb[j], preferred_element_type=jnp.float32)


def kernel(x, Q, P, WqT, WkT, Wgcn, bgcn):
    B, N, D = x.shape
    R = P.shape[0]
    K, _, reg_dim = Wgcn.shape
    Dq = WqT.shape[1]

    bt = _BT
    if B % bt or (bt * N) % 8:
        bt = B
    g = next((gg for gg in (_G, 2, 1) if B % (bt * gg) == 0), 1)
    S = B // (bt * g)

    bf = jnp.bfloat16
    x_flat = x.reshape(B * N, D)
    Q_tile = jnp.tile(Q, (bt, 1))
    eye_bt = jnp.eye(bt, dtype=jnp.float32)
    P_blk = jnp.kron(eye_bt, P.astype(jnp.float32))
    PT_blk = P_blk.T.astype(bf)
    P_blk = P_blk.astype(bf)
    Wqk = jnp.concatenate([WqT, WkT], axis=1).astype(bf)
    Wg = Wgcn.astype(bf)
    blk_mask = jnp.kron(eye_bt, jnp.ones((R, R), jnp.float32))
    bias = jnp.where(blk_mask > 0.5, 0.0, -1e30).astype(jnp.float32)

    kernel_fn = functools.partial(_fused_kernel, scale=1.0, k_order=K,
                                  dq=Dq, bt=bt, r_dim=R, g=g)

    out_shapes = (
        jax.ShapeDtypeStruct((B * N, reg_dim), jnp.float32),
        jax.ShapeDtypeStruct((B, R, R), jnp.float32),
    )

    grid_spec = pltpu.PrefetchScalarGridSpec(
        num_scalar_prefetch=0,
        grid=(S,),
        in_specs=[
            pl.BlockSpec((g * bt * N, D), lambda i: (i, 0)),
            pl.BlockSpec((bt * N, D), lambda i: (0, 0)),
            pl.BlockSpec((bt * R, bt * N), lambda i: (0, 0)),
            pl.BlockSpec((bt * N, bt * R), lambda i: (0, 0)),
            pl.BlockSpec((D, 2 * Dq), lambda i: (0, 0)),
            pl.BlockSpec((K, D, reg_dim), lambda i: (0, 0, 0)),
            pl.BlockSpec((1, reg_dim), lambda i: (0, 0)),
            pl.BlockSpec((bt * R, bt * R), lambda i: (0, 0)),
        ],
        out_specs=[
            pl.BlockSpec((g * bt * N, reg_dim), lambda i: (i, 0)),
            pl.BlockSpec((g * bt, R, R), lambda i: (i, 0, 0)),
        ],
    )

    r2n_flat, A_reg = pl.pallas_call(
        kernel_fn,
        grid_spec=grid_spec,
        out_shape=out_shapes,
        compiler_params=pltpu.CompilerParams(
            dimension_semantics=("parallel",)),
    )(x_flat, Q_tile, P_blk, PT_blk, Wqk, Wg, bgcn, bias)

    return r2n_flat.reshape(B, N, reg_dim), A_reg


# speedup vs baseline: 2.4461x; 1.2460x over previous
import functools
import jax
import jax.numpy as jnp
from jax import lax
from jax.experimental import pallas as pl
from jax.experimental.pallas import tpu as pltpu

_BT = 8
_G = 4


def _fused_kernel(x_ref, q_ref, pc_ref, ptr_ref, wqk_ref, gw_ref, gb_ref,
                  bias_ref, r2n_ref, attn_ref, *, scale, k_order, dq, bt,
                  r_dim, g):
    bf = jnp.bfloat16
    qn = q_ref[...]
    wqk = wqk_ref[...]
    bias = bias_ref[...]
    gb = gb_ref[...]
    gr = range(g)

    xf = [[jnp.maximum(qn * x_ref[j * bt + b], 0.0).astype(bf)
           for b in range(bt)] for j in gr]
    xr = [sum(jnp.dot(pc_ref[b], xf[j][b], preferred_element_type=jnp.float32)
              for b in range(bt)) for j in gr]
    xrb = [v.astype(bf) for v in xr]

    qk = [jnp.dot(xrb[j], wqk, preferred_element_type=jnp.float32)
          for j in gr]
    dots = [lax.dot_general(qk[j][:, :dq].astype(bf),
                            qk[j][:, dq:].astype(bf),
                            (((1,), (1,)), ((), ())),
                            preferred_element_type=jnp.float32)
            for j in gr]
    if scale != 1.0:
        dots = [d * scale for d in dots]
    dots = [d + bias for d in dots]
    mx = [jnp.max(d, axis=-1, keepdims=True) for d in dots]
    ex = [jnp.exp(dots[j] - mx[j]) for j in gr]
    attn = [ex[j] * pl.reciprocal(jnp.sum(ex[j], axis=-1, keepdims=True),
                                  approx=True) for j in gr]

    for j in gr:
        for b in range(bt):
            attn_ref[j * bt + b] = attn[j][b * r_dim:(b + 1) * r_dim,
                                          b * r_dim:(b + 1) * r_dim]

    attnb = [a.astype(bf) for a in attn]
    h = xrb
    out = [jnp.dot(h[j], gw_ref[0], preferred_element_type=jnp.float32)
           for j in gr]
    for kk in range(1, k_order):
        h = [jnp.dot(attnb[j], h[j],
                     preferred_element_type=jnp.float32).astype(bf)
             for j in gr]
        out = [out[j] + jnp.dot(h[j], gw_ref[kk],
                                preferred_element_type=jnp.float32)
               for j in gr]
    outb = [jnp.maximum(out[j] + gb, 0.0).astype(bf) for j in gr]

    for j in gr:
        for b in range(bt):
            r2n_ref[j * bt + b] = jnp.dot(
                ptr_ref[b], outb[j], preferred_element_type=jnp.float32)


def kernel(x, Q, P, WqT, WkT, Wgcn, bgcn):
    B, N, D = x.shape
    R = P.shape[0]
    K, _, reg_dim = Wgcn.shape
    Dq = WqT.shape[1]

    bt = _BT if B % _BT == 0 else 1
    g = next((gg for gg in (_G, 2, 1) if B % (bt * gg) == 0), 1)
    S = B // (bt * g)

    bf = jnp.bfloat16
    eye_bt = jnp.eye(bt, dtype=jnp.float32)
    P_blk = jnp.kron(eye_bt, P.astype(jnp.float32))
    P_cols = P_blk.reshape(bt * R, bt, N).transpose(1, 0, 2).astype(bf)
    PT_rows = P_blk.T.reshape(bt, N, bt * R).astype(bf)
    Wqk = jnp.concatenate([WqT, WkT], axis=1).astype(bf)
    Wg = Wgcn.astype(bf)
    blk_mask = jnp.kron(eye_bt, jnp.ones((R, R), jnp.float32))
    bias = jnp.where(blk_mask > 0.5, 0.0, -1e30).astype(jnp.float32)

    kernel_fn = functools.partial(_fused_kernel, scale=1.0, k_order=K,
                                  dq=Dq, bt=bt, r_dim=R, g=g)

    out_shapes = (
        jax.ShapeDtypeStruct((B, N, reg_dim), jnp.float32),
        jax.ShapeDtypeStruct((B, R, R), jnp.float32),
    )

    grid_spec = pltpu.PrefetchScalarGridSpec(
        num_scalar_prefetch=0,
        grid=(S,),
        in_specs=[
            pl.BlockSpec((g * bt, N, D), lambda i: (i, 0, 0)),
            pl.BlockSpec((N, D), lambda i: (0, 0)),
            pl.BlockSpec((bt, bt * R, N), lambda i: (0, 0, 0)),
            pl.BlockSpec((bt, N, bt * R), lambda i: (0, 0, 0)),
            pl.BlockSpec((D, 2 * Dq), lambda i: (0, 0)),
            pl.BlockSpec((K, D, reg_dim), lambda i: (0, 0, 0)),
            pl.BlockSpec((1, reg_dim), lambda i: (0, 0)),
            pl.BlockSpec((bt * R, bt * R), lambda i: (0, 0)),
        ],
        out_specs=[
            pl.BlockSpec((g * bt, N, reg_dim), lambda i: (i, 0, 0)),
            pl.BlockSpec((g * bt, R, R), lambda i: (i, 0, 0)),
        ],
    )

    reg2node, A_reg = pl.pallas_call(
        kernel_fn,
        grid_spec=grid_spec,
        out_shape=out_shapes,
        compiler_params=pltpu.CompilerParams(
            dimension_semantics=("parallel",)),
    )(x, Q, P_cols, PT_rows, Wqk, Wg, bgcn, bias)

    return reg2node, A_reg


# speedup vs baseline: 2.8643x; 1.1709x over previous
import functools
import numpy as np
import jax
import jax.numpy as jnp
from jax import lax
from jax.experimental import pallas as pl
from jax.experimental.pallas import tpu as pltpu

_BT = 8
_G = 4


def _fused_kernel(x_ref, q_ref, p_ref, wq_ref, wk_ref, gw_ref, gb_ref,
                  bias_ref, r2n_ref, attn_ref, *, scale, k_order, dq, bt,
                  r_dim, g):
    bf = jnp.bfloat16
    qn = q_ref[...]
    pbf = p_ref[...].astype(bf)
    wq = wq_ref[...].astype(bf)
    wk = wk_ref[...].astype(bf)
    g0 = gw_ref[0].astype(bf)
    bias = bias_ref[...]
    gb = gb_ref[...]
    gr = range(g)

    xf = [[jnp.maximum(qn * x_ref[j * bt + b], 0.0).astype(bf)
           for b in range(bt)] for j in gr]
    xr = [jnp.concatenate(
            [jnp.dot(pbf, xf[j][b], preferred_element_type=jnp.float32)
             for b in range(bt)], axis=0) for j in gr]
    xrb = [v.astype(bf) for v in xr]

    q = [jnp.dot(xrb[j], wq, preferred_element_type=jnp.float32).astype(bf)
         for j in gr]
    k = [jnp.dot(xrb[j], wk, preferred_element_type=jnp.float32).astype(bf)
         for j in gr]
    dots = [lax.dot_general(q[j], k[j], (((1,), (1,)), ((), ())),
                            preferred_element_type=jnp.float32)
            for j in gr]
    if scale != 1.0:
        dots = [d * scale for d in dots]
    dots = [d + bias for d in dots]
    mx = [jnp.max(d, axis=-1, keepdims=True) for d in dots]
    ex = [jnp.exp(dots[j] - mx[j]) for j in gr]
    attn = [ex[j] * pl.reciprocal(jnp.sum(ex[j], axis=-1, keepdims=True),
                                  approx=True) for j in gr]

    for j in gr:
        for b in range(bt):
            attn_ref[j * bt + b] = attn[j][b * r_dim:(b + 1) * r_dim,
                                          b * r_dim:(b + 1) * r_dim]

    attnb = [a.astype(bf) for a in attn]
    h = xrb
    out = [jnp.dot(h[j], g0, preferred_element_type=jnp.float32) for j in gr]
    for kk in range(1, k_order):
        gk = gw_ref[kk].astype(bf)
        h = [jnp.dot(attnb[j], h[j],
                     preferred_element_type=jnp.float32).astype(bf)
             for j in gr]
        out = [out[j] + jnp.dot(h[j], gk, preferred_element_type=jnp.float32)
               for j in gr]
    outb = [jnp.maximum(out[j] + gb, 0.0).astype(bf) for j in gr]

    for j in gr:
        for b in range(bt):
            r2n_ref[j * bt + b] = lax.dot_general(
                pbf, outb[j][b * r_dim:(b + 1) * r_dim],
                (((0,), (0,)), ((), ())),
                preferred_element_type=jnp.float32)


def kernel(x, Q, P, WqT, WkT, Wgcn, bgcn):
    B, N, D = x.shape
    R = P.shape[0]
    K, _, reg_dim = Wgcn.shape
    Dq = WqT.shape[1]

    bt = _BT if B % _BT == 0 else 1
    g = next((gg for gg in (_G, 2, 1) if B % (bt * gg) == 0), 1)
    S = B // (bt * g)

    blk = np.kron(np.eye(bt, dtype=np.float32), np.ones((R, R), np.float32))
    bias = jnp.asarray(np.where(blk > 0.5, 0.0, -1e30).astype(np.float32))

    kernel_fn = functools.partial(_fused_kernel, scale=1.0, k_order=K,
                                  dq=Dq, bt=bt, r_dim=R, g=g)

    out_shapes = (
        jax.ShapeDtypeStruct((B, N, reg_dim), jnp.float32),
        jax.ShapeDtypeStruct((B, R, R), jnp.float32),
    )

    grid_spec = pltpu.PrefetchScalarGridSpec(
        num_scalar_prefetch=0,
        grid=(S,),
        in_specs=[
            pl.BlockSpec((g * bt, N, D), lambda i: (i, 0, 0)),
            pl.BlockSpec((N, D), lambda i: (0, 0)),
            pl.BlockSpec((R, N), lambda i: (0, 0)),
            pl.BlockSpec((D, Dq), lambda i: (0, 0)),
            pl.BlockSpec((D, Dq), lambda i: (0, 0)),
            pl.BlockSpec((K, D, reg_dim), lambda i: (0, 0, 0)),
            pl.BlockSpec((1, reg_dim), lambda i: (0, 0)),
            pl.BlockSpec((bt * R, bt * R), lambda i: (0, 0)),
        ],
        out_specs=[
            pl.BlockSpec((g * bt, N, reg_dim), lambda i: (i, 0, 0)),
            pl.BlockSpec((g * bt, R, R), lambda i: (i, 0, 0)),
        ],
    )

    reg2node, A_reg = pl.pallas_call(
        kernel_fn,
        grid_spec=grid_spec,
        out_shape=out_shapes,
        compiler_params=pltpu.CompilerParams(
            dimension_semantics=("parallel",)),
    )(x, Q, P, WqT, WkT, Wgcn, bgcn, bias)

    return reg2node, A_reg


# speedup vs baseline: 3.4798x; 1.2149x over previous
import functools
import numpy as np
import jax
import jax.numpy as jnp
from jax import lax
from jax.experimental import pallas as pl
from jax.experimental.pallas import tpu as pltpu

_BT = 8
_G = 8


def _fused_kernel(x_ref, q_ref, p_ref, wq_ref, wk_ref, gw_ref, gb_ref,
                  bias_ref, r2n_ref, attn_ref, *, scale, k_order, bt,
                  r_dim, rp, g):
    bf = jnp.bfloat16
    qn = q_ref[...]
    pbf = p_ref[...].astype(bf)
    if rp > r_dim:
        p16 = jnp.concatenate(
            [pbf, jnp.zeros((rp - r_dim, pbf.shape[1]), bf)], axis=0)
    else:
        p16 = pbf
    wq = wq_ref[...].astype(bf)
    wk = wk_ref[...].astype(bf)
    g0 = gw_ref[0].astype(bf)
    bias = bias_ref[...]
    gb = gb_ref[...]
    gr = range(g)

    xf = [[jnp.maximum(qn * x_ref[j * bt + b], 0.0).astype(bf)
           for b in range(bt)] for j in gr]
    xr = [jnp.concatenate(
            [jnp.dot(p16, xf[j][b], preferred_element_type=jnp.float32)
             for b in range(bt)], axis=0) for j in gr]
    xrb = [v.astype(bf) for v in xr]

    q = [jnp.dot(xrb[j], wq, preferred_element_type=jnp.float32).astype(bf)
         for j in gr]
    k = [jnp.dot(xrb[j], wk, preferred_element_type=jnp.float32).astype(bf)
         for j in gr]
    dots = [lax.dot_general(q[j], k[j], (((1,), (1,)), ((), ())),
                            preferred_element_type=jnp.float32)
            for j in gr]
    if scale != 1.0:
        dots = [d * scale for d in dots]
    dots = [d + bias for d in dots]
    mx = [jnp.max(d, axis=-1, keepdims=True) for d in dots]
    ex = [jnp.exp(dots[j] - mx[j]) for j in gr]
    attn = [ex[j] * pl.reciprocal(jnp.sum(ex[j], axis=-1, keepdims=True),
                                  approx=True) for j in gr]

    for j in gr:
        for b in range(bt):
            attn_ref[j * bt + b] = attn[j][b * rp:b * rp + r_dim,
                                          b * rp:b * rp + r_dim]

    attnb = [a.astype(bf) for a in attn]
    h = xrb
    out = [jnp.dot(h[j], g0, preferred_element_type=jnp.float32) for j in gr]
    for kk in range(1, k_order):
        gk = gw_ref[kk].astype(bf)
        h = [jnp.dot(attnb[j], h[j],
                     preferred_element_type=jnp.float32).astype(bf)
             for j in gr]
        out = [out[j] + jnp.dot(h[j], gk, preferred_element_type=jnp.float32)
               for j in gr]
    outb = [jnp.maximum(out[j] + gb, 0.0).astype(bf) for j in gr]

    for j in gr:
        for b in range(bt):
            r2n_ref[j * bt + b] = lax.dot_general(
                pbf, outb[j][b * rp:b * rp + r_dim],
                (((0,), (0,)), ((), ())),
                preferred_element_type=jnp.float32)


def kernel(x, Q, P, WqT, WkT, Wgcn, bgcn):
    B, N, D = x.shape
    R = P.shape[0]
    K, _, reg_dim = Wgcn.shape

    bt = _BT if B % _BT == 0 else 1
    g = next((gg for gg in (_G, 4, 2, 1) if B % (bt * gg) == 0), 1)
    S = B // (bt * g)
    rp = -(-R // 8) * 8

    idx = np.arange(bt * rp)
    same_blk = (idx[:, None] // rp) == (idx[None, :] // rp)
    valid = ((idx[:, None] % rp) < R) & ((idx[None, :] % rp) < R)
    bias = jnp.asarray(np.where(same_blk & valid, 0.0, -1e30)
                       .astype(np.float32))

    kernel_fn = functools.partial(_fused_kernel, scale=1.0, k_order=K,
                                  bt=bt, r_dim=R, rp=rp, g=g)

    out_shapes = (
        jax.ShapeDtypeStruct((B, N, reg_dim), jnp.float32),
        jax.ShapeDtypeStruct((B, R, R), jnp.float32),
    )

    grid_spec = pltpu.PrefetchScalarGridSpec(
        num_scalar_prefetch=0,
        grid=(S,),
        in_specs=[
            pl.BlockSpec((g * bt, N, D), lambda i: (i, 0, 0)),
            pl.BlockSpec((N, D), lambda i: (0, 0)),
            pl.BlockSpec((R, N), lambda i: (0, 0)),
            pl.BlockSpec((D, WqT.shape[1]), lambda i: (0, 0)),
            pl.BlockSpec((D, WkT.shape[1]), lambda i: (0, 0)),
            pl.BlockSpec((K, D, reg_dim), lambda i: (0, 0, 0)),
            pl.BlockSpec((1, reg_dim), lambda i: (0, 0)),
            pl.BlockSpec((bt * rp, bt * rp), lambda i: (0, 0)),
        ],
        out_specs=[
            pl.BlockSpec((g * bt, N, reg_dim), lambda i: (i, 0, 0)),
            pl.BlockSpec((g * bt, R, R), lambda i: (i, 0, 0)),
        ],
    )

    reg2node, A_reg = pl.pallas_call(
        kernel_fn,
        grid_spec=grid_spec,
        out_shape=out_shapes,
        compiler_params=pltpu.CompilerParams(
            dimension_semantics=("parallel",)),
    )(x, Q, P, WqT, WkT, Wgcn, bgcn, bias)

    return reg2node, A_reg


# speedup vs baseline: 3.5660x; 1.0248x over previous
import functools
import numpy as np
import jax
import jax.numpy as jnp
from jax import lax
from jax.experimental import pallas as pl
from jax.experimental.pallas import tpu as pltpu

_BT = 8
_G = 8


def _fused_kernel(x_ref, q_ref, p_ref, wq_ref, wk_ref, gw_ref, gb_ref,
                  bias_ref, r2n_ref, attn_ref, pm_ref, *, scale, k_order,
                  bt, r_dim, rp, n_dim, npad, g):
    bf = jnp.bfloat16
    f32 = jnp.float32
    qn = q_ref[...]
    gb = gb_ref[...]
    bias = bias_ref[...]
    gr = range(g)

    pbf = p_ref[...].astype(bf)
    pm_ref[...] = jnp.zeros((bt * rp, bt * npad), bf)
    for b in range(bt):
        pm_ref[b * rp:b * rp + r_dim, b * npad:b * npad + n_dim] = pbf
    pm = pm_ref[...]

    m_mat = lax.dot_general(wq_ref[...].astype(bf), wk_ref[...].astype(bf),
                            (((1,), (1,)), ((), ())),
                            preferred_element_type=f32).astype(bf)
    g0 = gw_ref[0].astype(bf)

    zrow = jnp.zeros((npad - n_dim, qn.shape[1]), bf)
    xfm = [jnp.concatenate(
             [v for b in range(bt)
              for v in (jnp.maximum(qn * x_ref[j * bt + b], 0.0).astype(bf),
                        zrow)], axis=0) for j in gr]
    xr = [jnp.dot(pm, xfm[j], preferred_element_type=f32) for j in gr]
    xrb = [v.astype(bf) for v in xr]

    xrm = [jnp.dot(xrb[j], m_mat, preferred_element_type=f32).astype(bf)
           for j in gr]
    dots = [lax.dot_general(xrm[j], xrb[j], (((1,), (1,)), ((), ())),
                            preferred_element_type=f32) for j in gr]
    if scale != 1.0:
        dots = [d * scale for d in dots]
    dots = [d + bias for d in dots]
    mx = [jnp.max(d, axis=-1, keepdims=True) for d in dots]
    ex = [jnp.exp(dots[j] - mx[j]) for j in gr]
    attn = [ex[j] * pl.reciprocal(jnp.sum(ex[j], axis=-1, keepdims=True),
                                  approx=True) for j in gr]

    for j in gr:
        for b in range(bt):
            attn_ref[j * bt + b] = attn[j][b * rp:b * rp + r_dim,
                                          b * rp:b * rp + r_dim]

    attnb = [a.astype(bf) for a in attn]
    h = xrb
    out = [jnp.dot(h[j], g0, preferred_element_type=f32) for j in gr]
    for kk in range(1, k_order):
        gk = gw_ref[kk].astype(bf)
        hg = [jnp.dot(h[j], gk, preferred_element_type=f32).astype(bf)
              for j in gr]
        out = [out[j] + jnp.dot(attnb[j], hg[j], preferred_element_type=f32)
               for j in gr]
        if kk + 1 < k_order:
            h = [jnp.dot(attnb[j], h[j],
                         preferred_element_type=f32).astype(bf) for j in gr]
    outb = [jnp.maximum(out[j] + gb, 0.0).astype(bf) for j in gr]

    for j in gr:
        r2n = lax.dot_general(pm, outb[j], (((0,), (0,)), ((), ())),
                              preferred_element_type=f32)
        for b in range(bt):
            r2n_ref[j * bt + b] = r2n[b * npad:b * npad + n_dim]


def kernel(x, Q, P, WqT, WkT, Wgcn, bgcn):
    B, N, D = x.shape
    R = P.shape[0]
    K, _, reg_dim = Wgcn.shape

    bt = _BT if B % _BT == 0 else 1
    g = next((gg for gg in (_G, 4, 2, 1) if B % (bt * gg) == 0), 1)
    S = B // (bt * g)
    rp = -(-R // 8) * 8
    npad = -(-N // 8) * 8

    idx = np.arange(bt * rp)
    same_blk = (idx[:, None] // rp) == (idx[None, :] // rp)
    valid = ((idx[:, None] % rp) < R) & ((idx[None, :] % rp) < R)
    bias = jnp.asarray(np.where(same_blk & valid, 0.0, -1e30)
                       .astype(np.float32))

    kernel_fn = functools.partial(_fused_kernel, scale=1.0, k_order=K,
                                  bt=bt, r_dim=R, rp=rp, n_dim=N,
                                  npad=npad, g=g)

    out_shapes = (
        jax.ShapeDtypeStruct((B, N, reg_dim), jnp.float32),
        jax.ShapeDtypeStruct((B, R, R), jnp.float32),
    )

    grid_spec = pltpu.PrefetchScalarGridSpec(
        num_scalar_prefetch=0,
        grid=(S,),
        in_specs=[
            pl.BlockSpec((g * bt, N, D), lambda i: (i, 0, 0)),
            pl.BlockSpec((N, D), lambda i: (0, 0)),
            pl.BlockSpec((R, N), lambda i: (0, 0)),
            pl.BlockSpec((D, WqT.shape[1]), lambda i: (0, 0)),
            pl.BlockSpec((D, WkT.shape[1]), lambda i: (0, 0)),
            pl.BlockSpec((K, D, reg_dim), lambda i: (0, 0, 0)),
            pl.BlockSpec((1, reg_dim), lambda i: (0, 0)),
            pl.BlockSpec((bt * rp, bt * rp), lambda i: (0, 0)),
        ],
        out_specs=[
            pl.BlockSpec((g * bt, N, reg_dim), lambda i: (i, 0, 0)),
            pl.BlockSpec((g * bt, R, R), lambda i: (i, 0, 0)),
        ],
        scratch_shapes=[pltpu.VMEM((bt * rp, bt * npad), jnp.bfloat16)],
    )

    reg2node, A_reg = pl.pallas_call(
        kernel_fn,
        grid_spec=grid_spec,
        out_shape=out_shapes,
        compiler_params=pltpu.CompilerParams(
            dimension_semantics=("parallel",)),
    )(x, Q, P, WqT, WkT, Wgcn, bgcn, bias)

    return reg2node, A_reg


# speedup vs baseline: 3.8701x; 1.0853x over previous
import functools
import numpy as np
import jax
import jax.numpy as jnp
from jax import lax
from jax.experimental import pallas as pl
from jax.experimental.pallas import tpu as pltpu

_BT = 8
_G = 16


def _fused_kernel(x_ref, q_ref, p_ref, wq_ref, wk_ref, gw_ref, gb_ref,
                  bias_ref, r2n_ref, attn_ref, pm_ref, *, scale, k_order,
                  bt, r_dim, rp, n_dim, npad, g):
    bf = jnp.bfloat16
    f32 = jnp.float32
    qn = q_ref[...]
    gb = gb_ref[...]
    bias = bias_ref[...]
    gr = range(g)

    pbf = p_ref[...].astype(bf)
    pm_ref[...] = jnp.zeros((bt * rp, bt * npad), bf)
    for b in range(bt):
        pm_ref[b * rp:b * rp + r_dim, b * npad:b * npad + n_dim] = pbf
    pm = pm_ref[...]

    m_mat = lax.dot_general(wq_ref[...].astype(bf), wk_ref[...].astype(bf),
                            (((1,), (1,)), ((), ())),
                            preferred_element_type=f32).astype(bf)
    g0 = gw_ref[0].astype(bf)

    zrow = jnp.zeros((npad - n_dim, qn.shape[1]), bf)
    xfm = [jnp.concatenate(
             [v for b in range(bt)
              for v in (jnp.maximum(qn * x_ref[j * bt + b], 0.0).astype(bf),
                        zrow)], axis=0) for j in gr]
    xr = [jnp.dot(pm, xfm[j], preferred_element_type=f32) for j in gr]
    xrb = [v.astype(bf) for v in xr]

    xrm = [jnp.dot(xrb[j], m_mat, preferred_element_type=f32).astype(bf)
           for j in gr]
    dots = [lax.dot_general(xrm[j], xrb[j], (((1,), (1,)), ((), ())),
                            preferred_element_type=f32) for j in gr]
    if scale != 1.0:
        dots = [d * scale for d in dots]
    dots = [d + bias for d in dots]
    mx = [jnp.max(d, axis=-1, keepdims=True) for d in dots]
    ex = [jnp.exp(dots[j] - mx[j]) for j in gr]
    attn = [ex[j] * pl.reciprocal(jnp.sum(ex[j], axis=-1, keepdims=True),
                                  approx=True) for j in gr]

    for j in gr:
        for b in range(bt):
            attn_ref[j * bt + b] = attn[j][b * rp:b * rp + r_dim,
                                          b * rp:b * rp + r_dim]

    attnb = [a.astype(bf) for a in attn]
    h = xrb
    out = [jnp.dot(h[j], g0, preferred_element_type=f32) for j in gr]
    for kk in range(1, k_order):
        gk = gw_ref[kk].astype(bf)
        hg = [jnp.dot(h[j], gk, preferred_element_type=f32).astype(bf)
              for j in gr]
        out = [out[j] + jnp.dot(attnb[j], hg[j], preferred_element_type=f32)
               for j in gr]
        if kk + 1 < k_order:
            h = [jnp.dot(attnb[j], h[j],
                         preferred_element_type=f32).astype(bf) for j in gr]
    outb = [jnp.maximum(out[j] + gb, 0.0).astype(bf) for j in gr]

    for j in gr:
        r2n = lax.dot_general(pm, outb[j], (((0,), (0,)), ((), ())),
                              preferred_element_type=f32)
        for b in range(bt):
            r2n_ref[j * bt + b] = r2n[b * npad:b * npad + n_dim]


def kernel(x, Q, P, WqT, WkT, Wgcn, bgcn):
    B, N, D = x.shape
    R = P.shape[0]
    K, _, reg_dim = Wgcn.shape

    bt = _BT if B % _BT == 0 else 1
    g = next((gg for gg in (_G, 8, 4, 2, 1) if B % (bt * gg) == 0), 1)
    S = B // (bt * g)
    rp = -(-R // 8) * 8
    npad = -(-N // 8) * 8

    idx = np.arange(bt * rp)
    same_blk = (idx[:, None] // rp) == (idx[None, :] // rp)
    valid = ((idx[:, None] % rp) < R) & ((idx[None, :] % rp) < R)
    bias = jnp.asarray(np.where(same_blk & valid, 0.0, -1e30)
                       .astype(np.float32))

    kernel_fn = functools.partial(_fused_kernel, scale=1.0, k_order=K,
                                  bt=bt, r_dim=R, rp=rp, n_dim=N,
                                  npad=npad, g=g)

    out_shapes = (
        jax.ShapeDtypeStruct((B, N, reg_dim), jnp.float32),
        jax.ShapeDtypeStruct((B, R, R), jnp.float32),
    )

    grid_spec = pltpu.PrefetchScalarGridSpec(
        num_scalar_prefetch=0,
        grid=(S,),
        in_specs=[
            pl.BlockSpec((g * bt, N, D), lambda i: (i, 0, 0)),
            pl.BlockSpec((N, D), lambda i: (0, 0)),
            pl.BlockSpec((R, N), lambda i: (0, 0)),
            pl.BlockSpec((D, WqT.shape[1]), lambda i: (0, 0)),
            pl.BlockSpec((D, WkT.shape[1]), lambda i: (0, 0)),
            pl.BlockSpec((K, D, reg_dim), lambda i: (0, 0, 0)),
            pl.BlockSpec((1, reg_dim), lambda i: (0, 0)),
            pl.BlockSpec((bt * rp, bt * rp), lambda i: (0, 0)),
        ],
        out_specs=[
            pl.BlockSpec((g * bt, N, reg_dim), lambda i: (i, 0, 0)),
            pl.BlockSpec((g * bt, R, R), lambda i: (i, 0, 0)),
        ],
        scratch_shapes=[pltpu.VMEM((bt * rp, bt * npad), jnp.bfloat16)],
    )

    reg2node, A_reg = pl.pallas_call(
        kernel_fn,
        grid_spec=grid_spec,
        out_shape=out_shapes,
        compiler_params=pltpu.CompilerParams(
            dimension_semantics=("parallel",)),
    )(x, Q, P, WqT, WkT, Wgcn, bgcn, bias)

    return reg2node, A_reg


# speedup vs baseline: 4.0476x; 1.0459x over previous
import functools
import numpy as np
import jax
import jax.numpy as jnp
from jax import lax
from jax.experimental import pallas as pl
from jax.experimental.pallas import tpu as pltpu

_BT = 8
_G = 32


def _fused_kernel(x_ref, q_ref, p_ref, wq_ref, wk_ref, gw_ref, gb_ref,
                  bias_ref, r2n_ref, attn_ref, pm_ref, *, scale, k_order,
                  bt, r_dim, rp, n_dim, npad, g):
    bf = jnp.bfloat16
    f32 = jnp.float32
    qn = q_ref[...]
    gb = gb_ref[...]
    bias = bias_ref[...]
    gr = range(g)

    pbf = p_ref[...].astype(bf)
    pm_ref[...] = jnp.zeros((bt * rp, bt * npad), bf)
    for b in range(bt):
        pm_ref[b * rp:b * rp + r_dim, b * npad:b * npad + n_dim] = pbf
    pm = pm_ref[...]

    m_mat = lax.dot_general(wq_ref[...].astype(bf), wk_ref[...].astype(bf),
                            (((1,), (1,)), ((), ())),
                            preferred_element_type=f32).astype(bf)
    g0 = gw_ref[0].astype(bf)

    zrow = jnp.zeros((npad - n_dim, qn.shape[1]), bf)
    xfm = [jnp.concatenate(
             [v for b in range(bt)
              for v in (jnp.maximum(qn * x_ref[j * bt + b], 0.0).astype(bf),
                        zrow)], axis=0) for j in gr]
    xr = [jnp.dot(pm, xfm[j], preferred_element_type=f32) for j in gr]
    xrb = [v.astype(bf) for v in xr]

    xrm = [jnp.dot(xrb[j], m_mat, preferred_element_type=f32).astype(bf)
           for j in gr]
    dots = [lax.dot_general(xrm[j], xrb[j], (((1,), (1,)), ((), ())),
                            preferred_element_type=f32) for j in gr]
    if scale != 1.0:
        dots = [d * scale for d in dots]
    dots = [d + bias for d in dots]
    mx = [jnp.max(d, axis=-1, keepdims=True) for d in dots]
    ex = [jnp.exp(dots[j] - mx[j]) for j in gr]
    attn = [ex[j] * pl.reciprocal(jnp.sum(ex[j], axis=-1, keepdims=True),
                                  approx=True) for j in gr]

    for j in gr:
        for b in range(bt):
            attn_ref[j * bt + b] = attn[j][b * rp:b * rp + r_dim,
                                          b * rp:b * rp + r_dim]

    attnb = [a.astype(bf) for a in attn]
    h = xrb
    out = [jnp.dot(h[j], g0, preferred_element_type=f32) for j in gr]
    for kk in range(1, k_order):
        gk = gw_ref[kk].astype(bf)
        hg = [jnp.dot(h[j], gk, preferred_element_type=f32).astype(bf)
              for j in gr]
        out = [out[j] + jnp.dot(attnb[j], hg[j], preferred_element_type=f32)
               for j in gr]
        if kk + 1 < k_order:
            h = [jnp.dot(attnb[j], h[j],
                         preferred_element_type=f32).astype(bf) for j in gr]
    outb = [jnp.maximum(out[j] + gb, 0.0).astype(bf) for j in gr]

    for j in gr:
        r2n = lax.dot_general(pm, outb[j], (((0,), (0,)), ((), ())),
                              preferred_element_type=f32)
        for b in range(bt):
            r2n_ref[j * bt + b] = r2n[b * npad:b * npad + n_dim]


def kernel(x, Q, P, WqT, WkT, Wgcn, bgcn):
    B, N, D = x.shape
    R = P.shape[0]
    K, _, reg_dim = Wgcn.shape

    bt = _BT if B % _BT == 0 else 1
    g = next((gg for gg in (_G, 16, 8, 4, 2, 1) if B % (bt * gg) == 0), 1)
    S = B // (bt * g)
    rp = -(-R // 8) * 8
    npad = -(-N // 8) * 8

    idx = np.arange(bt * rp)
    same_blk = (idx[:, None] // rp) == (idx[None, :] // rp)
    valid = ((idx[:, None] % rp) < R) & ((idx[None, :] % rp) < R)
    bias = jnp.asarray(np.where(same_blk & valid, 0.0, -1e30)
                       .astype(np.float32))

    kernel_fn = functools.partial(_fused_kernel, scale=1.0, k_order=K,
                                  bt=bt, r_dim=R, rp=rp, n_dim=N,
                                  npad=npad, g=g)

    out_shapes = (
        jax.ShapeDtypeStruct((B, N, reg_dim), jnp.float32),
        jax.ShapeDtypeStruct((B, R, R), jnp.float32),
    )

    grid_spec = pltpu.PrefetchScalarGridSpec(
        num_scalar_prefetch=0,
        grid=(S,),
        in_specs=[
            pl.BlockSpec((g * bt, N, D), lambda i: (i, 0, 0)),
            pl.BlockSpec((N, D), lambda i: (0, 0)),
            pl.BlockSpec((R, N), lambda i: (0, 0)),
            pl.BlockSpec((D, WqT.shape[1]), lambda i: (0, 0)),
            pl.BlockSpec((D, WkT.shape[1]), lambda i: (0, 0)),
            pl.BlockSpec((K, D, reg_dim), lambda i: (0, 0, 0)),
            pl.BlockSpec((1, reg_dim), lambda i: (0, 0)),
            pl.BlockSpec((bt * rp, bt * rp), lambda i: (0, 0)),
        ],
        out_specs=[
            pl.BlockSpec((g * bt, N, reg_dim), lambda i: (i, 0, 0)),
            pl.BlockSpec((g * bt, R, R), lambda i: (i, 0, 0)),
        ],
        scratch_shapes=[pltpu.VMEM((bt * rp, bt * npad), jnp.bfloat16)],
    )

    reg2node, A_reg = pl.pallas_call(
        kernel_fn,
        grid_spec=grid_spec,
        out_shape=out_shapes,
        compiler_params=pltpu.CompilerParams(
            dimension_semantics=("parallel",)),
    )(x, Q, P, WqT, WkT, Wgcn, bgcn, bias)

    return reg2node, A_reg
